# Initial kernel scaffold; baseline (speedup 1.0000x reference)
import jax
import jax.numpy as jnp
from jax.experimental import pallas as pl


def kernel(x, edge_index_1, edge_index_2, params):
    raise NotImplementedError("write your pallas kernel here")


# speedup vs baseline: 2.1079x; 2.1079x over previous
import functools

import jax
import jax.numpy as jnp
from jax import lax
from jax.experimental import pallas as pl
from jax.experimental.pallas import tpu as pltpu
from jax.experimental.pallas import tpu_sc as plsc

N = 10000
DH = 128
NB = 1000
CHUNK = 128
SB = 8
NSUB = 16
SEED_ROWS = 632
ACC_ROWS = N + 8

_pallas_call = pl.pallas_call


def _make_segsum(epad, T):
    nsb = epad // (CHUNK * SB * NSUB)
    mesh = plsc.VectorSubcoreMesh(core_axis_name="c", subcore_axis_name="s")

    @functools.partial(
        pl.kernel,
        mesh=mesh,
        out_type=jax.ShapeDtypeStruct((2, T, N, DH), jnp.float32),
        scratch_types=[
            pltpu.VMEM((SB, CHUNK), jnp.int32),
            pltpu.VMEM((SB, CHUNK), jnp.int32),
            pltpu.VMEM((CHUNK, DH), jnp.float32),
            pltpu.VMEM((CHUNK, DH), jnp.float32),
            pltpu.VMEM_SHARED((ACC_ROWS, DH), jnp.float32),
            pltpu.SemaphoreType.DMA,
            pltpu.SemaphoreType.DMA,
        ],
    )
    def segsum(tables, e1s, e1d, e2s, e2d, out,
               srcbuf, dstbuf, rows0, rows1, acc, sem0, sem1):
        c = lax.axis_index("c")
        s = lax.axis_index("s")
        row0 = jnp.minimum(s * SEED_ROWS, N - SEED_ROWS)
        rows = (rows0, rows1)
        sems = (sem0, sem1)
        for r in range(T // 2):
            t = 2 * r + c
            tbl = tables.at[t]
            for v, (src_hbm, dst_hbm) in enumerate(((e1s, e1d), (e2s, e2d))):
                pltpu.sync_copy(tbl.at[pl.ds(row0, SEED_ROWS)],
                                acc.at[pl.ds(row0, SEED_ROWS)])
                plsc.subcore_barrier()

                def body(j, carry):
                    idx0 = (s * nsb + j) * SB
                    pltpu.sync_copy(src_hbm.at[pl.ds(idx0, SB)], srcbuf)
                    pltpu.sync_copy(dst_hbm.at[pl.ds(idx0, SB)], dstbuf)
                    cp = pltpu.async_copy(tbl.at[srcbuf.at[0]], rows[0], sems[0])
                    for k in range(SB):
                        cp.wait()
                        if k + 1 < SB:
                            cp = pltpu.async_copy(
                                tbl.at[srcbuf.at[k + 1]],
                                rows[(k + 1) % 2], sems[(k + 1) % 2])
                        pltpu.sync_copy(rows[k % 2], acc.at[dstbuf.at[k]],
                                        add=True)
                    return carry

                lax.fori_loop(0, nsb, body, 0)
                plsc.subcore_barrier()
                pltpu.sync_copy(
                    acc.at[pl.ds(row0, SEED_ROWS)],
                    out.at[v, t, pl.ds(row0, SEED_ROWS)],
                )
                plsc.subcore_barrier()

    return segsum


def _proj_body(h_ref, w_ref, o_ref):
    T = h_ref.shape[1]
    hh = jnp.concatenate([h_ref[0, t] for t in range(T)], axis=1)
    o_ref[0] = jnp.dot(hh, w_ref[0], preferred_element_type=jnp.float32)


def _proj_matmul(h, w4):
    T = h.shape[1]
    return _pallas_call(
        _proj_body,
        grid=(N // NB, 4),
        in_specs=[
            pl.BlockSpec((1, T, NB, DH), lambda i, j: (j // 2, 0, i, 0)),
            pl.BlockSpec((1, T * DH, DH), lambda i, j: (j, 0, 0)),
        ],
        out_specs=pl.BlockSpec((1, NB, DH), lambda i, j: (j, i, 0)),
        out_shape=jax.ShapeDtypeStruct((4, N, DH), jnp.float32),
    )(h, w4)


def _sum_body(h_ref, s_ref):
    i = pl.program_id(1)
    h = h_ref[0]
    ps = jnp.broadcast_to(jnp.sum(h, axis=0, keepdims=True), (8, DH))

    @pl.when(i == 0)
    def _():
        s_ref[0] = ps

    @pl.when(i != 0)
    def _():
        s_ref[0] += ps


def _mean(y4):
    s8 = _pallas_call(
        _sum_body,
        grid=(4, N // NB),
        in_specs=[pl.BlockSpec((1, NB, DH), lambda j, i: (j, i, 0))],
        out_specs=pl.BlockSpec((1, 8, DH), lambda j, i: (j, 0, 0)),
        out_shape=jax.ShapeDtypeStruct((4, 8, DH), jnp.float32),
    )(y4)
    return s8[:, 0, :] / N


def _var_body(h_ref, m_ref, q_ref):
    i = pl.program_id(1)
    d = h_ref[0] - m_ref[0, 0:1, :]
    pq = jnp.broadcast_to(jnp.sum(d * d, axis=0, keepdims=True), (8, DH))

    @pl.when(i == 0)
    def _():
        q_ref[0] = pq

    @pl.when(i != 0)
    def _():
        q_ref[0] += pq


def _var(y4, m4):
    m8 = jnp.broadcast_to(m4[:, None, :], (4, 8, DH))
    q8 = _pallas_call(
        _var_body,
        grid=(4, N // NB),
        in_specs=[
            pl.BlockSpec((1, NB, DH), lambda j, i: (j, i, 0)),
            pl.BlockSpec((1, 8, DH), lambda j, i: (j, 0, 0)),
        ],
        out_specs=pl.BlockSpec((1, 8, DH), lambda j, i: (j, 0, 0)),
        out_shape=jax.ShapeDtypeStruct((4, 8, DH), jnp.float32),
    )(y4, m8)
    return q8[:, 0, :] / N


def _tail_body(h_ref, m_ref, s_ref, g_ref, be_ref, w2a_ref, w2b_ref, b2_ref,
               wa_ref, ba_ref, wb_ref, bb_ref, o_ref):
    h = h_ref[...]
    cat = lambda r, v: jnp.concatenate(
        [r[2 * v:2 * v + 1], r[2 * v + 1:2 * v + 2]], axis=1)
    xs = []
    for v in range(2):
        z = jnp.concatenate([h[2 * v], h[2 * v + 1]], axis=1)
        z = cat(g_ref, v) * (z - cat(m_ref, v)) / cat(s_ref, v) + cat(be_ref, v)
        z = jnp.maximum(z, 0.0)
        w2 = (w2a_ref, w2b_ref)[v][...]
        b2 = b2_ref[0:1, v * 256:(v + 1) * 256]
        xs.append(jnp.maximum(
            jnp.dot(z, w2, preferred_element_type=jnp.float32) + b2, 0.0))
    t = jnp.dot(xs[0], wa_ref[0:256], preferred_element_type=jnp.float32)
    t = t + jnp.dot(xs[1], wa_ref[256:512], preferred_element_type=jnp.float32)
    t = jnp.maximum(t + ba_ref[0:1, :], 0.0)
    o_ref[...] = jnp.dot(t, wb_ref[...], preferred_element_type=jnp.float32) + bb_ref[0:1, :]


def _tail(y4, m8, s8, g8, be8, w2a, w2b, b2cat, wa, ba2, wb, bb2):
    full = lambda i: (0, 0)
    return _pallas_call(
        _tail_body,
        grid=(N // NB,),
        in_specs=[
            pl.BlockSpec((4, NB, DH), lambda i: (0, i, 0)),
            pl.BlockSpec((8, DH), full),
            pl.BlockSpec((8, DH), full),
            pl.BlockSpec((8, DH), full),
            pl.BlockSpec((8, DH), full),
            pl.BlockSpec((256, 256), full),
            pl.BlockSpec((256, 256), full),
            pl.BlockSpec((8, 512), full),
            pl.BlockSpec((512, 256), full),
            pl.BlockSpec((8, 256), full),
            pl.BlockSpec((256, 256), full),
            pl.BlockSpec((8, 256), full),
        ],
        out_specs=pl.BlockSpec((NB, 256), lambda i: (i, 0)),
        out_shape=jax.ShapeDtypeStruct((N, 256), jnp.float32),
    )(y4, m8, s8, g8, be8, w2a, w2b, b2cat, wa, ba2, wb, bb2)


def _halves(v):
    return jnp.stack([v[:DH], v[DH:]])


def _layer(tables, edges, segsum, pa, pb, pm):
    T = tables.shape[0]
    h = segsum(tables, *edges)
    kdim = T * DH
    w1a = pa["W1"]
    w1b = pb["W1"]
    if w1a.shape[0] < kdim:
        zp = jnp.zeros((kdim - w1a.shape[0], w1a.shape[1]), jnp.float32)
        w1a = jnp.concatenate([w1a, zp], axis=0)
        w1b = jnp.concatenate([w1b, zp], axis=0)
    w4 = jnp.stack([w1a[:, :DH], w1a[:, DH:], w1b[:, :DH], w1b[:, DH:]])
    y4 = _proj_matmul(h, w4)
    m4 = _mean(y4)
    v4 = _var(y4, m4)
    s4 = jnp.sqrt(v4 + 1e-5)
    g4 = jnp.concatenate([_halves(pa["g"]), _halves(pb["g"])])
    be4 = jnp.concatenate([_halves(pa["be"]), _halves(pb["be"])])
    pad8 = lambda a: jnp.concatenate([a, jnp.zeros((4, DH), jnp.float32)], axis=0)
    b2cat = jnp.broadcast_to(
        jnp.concatenate([pa["b2"], pb["b2"]])[None, :], (8, 512))
    ba2 = jnp.broadcast_to(pm["ba"][None, :], (8, 256))
    bb2 = jnp.broadcast_to(pm["bb"][None, :], (8, 256))
    return _tail(y4, pad8(m4), pad8(s4), pad8(g4), pad8(be4),
                 pa["W2"], pb["W2"], b2cat, pm["Wa"], ba2, pm["Wb"], bb2)


def _pad_idx(idx, epad, fill):
    pad = epad - idx.shape[0]
    padded = jnp.concatenate(
        [idx.astype(jnp.int32), jnp.full((pad,), fill, jnp.int32)])
    return padded.reshape(epad // CHUNK, CHUNK)


def kernel(x, edge_index_1, edge_index_2, params):
    e = edge_index_1.shape[1]
    step = NSUB * CHUNK * SB
    epad = ((e + step - 1) // step) * step
    edges = (
        _pad_idx(edge_index_1[0], epad, 0),
        _pad_idx(edge_index_1[1], epad, N),
        _pad_idx(edge_index_2[0], epad, 0),
        _pad_idx(edge_index_2[1], epad, N),
    )
    p = params

    f_in = x.shape[1]
    t1 = -(-f_in // DH)
    xpad = jnp.pad(x, ((0, 0), (0, t1 * DH - f_in)))
    tables = jnp.transpose(xpad.reshape(N, t1, DH), (1, 0, 2))
    h = _layer(tables, edges, _make_segsum(epad, t1),
               p["conv_1_1"], p["conv_1_2"], p["mlp_1"])

    seg2 = _make_segsum(epad, 2)
    for pa, pb, pm in (
        (p["conv_2_1"], p["conv_2_2"], p["mlp_2"]),
        (p["conv_3_1"], p["conv_3_2"], p["mlp_2"]),
    ):
        tables = jnp.stack([h[:, :DH], h[:, DH:]])
        h = _layer(tables, edges, seg2, pa, pb, pm)
    return h
